# Initial kernel scaffold; baseline (speedup 1.0000x reference)
import jax
import jax.numpy as jnp
from jax.experimental import pallas as pl


def kernel(s, x, adj, train_ids, W1, b1, W2, b2, Wl1, bl1, Wl2, bl2):
    raise NotImplementedError("write your pallas kernel here")


# speedup vs baseline: 2.8717x; 2.8717x over previous
import functools

import jax
import jax.numpy as jnp
from jax import lax
from jax.experimental import pallas as pl
from jax.experimental.pallas import tpu as pltpu
from jax.experimental.pallas import tpu_sc as plsc

N = 10000
NFEAT = 128
NHID = 128
NS = 16
E = 160000

BM = 400


def _dense_body(adj_ref, x_ref, s_ref, w1_ref, b1_ref, w2_ref, b2_ref,
                wa_ref, wsa_ref, wb_ref, wsb_ref, bl1_ref,
                a_ref, b_ref, g_scr):
    l = pl.program_id(0)
    i = pl.program_id(1)

    @pl.when(l == 0)
    def _():
        ax = jnp.dot(adj_ref[...], x_ref[...],
                     preferred_element_type=jnp.float32)
        h = jnp.dot(ax, w1_ref[...], preferred_element_type=jnp.float32)
        h = jnp.maximum(h + b1_ref[...], 0.0)
        g_scr[pl.ds(i * BM, BM), :] = jnp.dot(
            h, w2_ref[...], preferred_element_type=jnp.float32)

    @pl.when(l == 1)
    def _():
        h2 = jnp.dot(adj_ref[...], g_scr[...],
                     preferred_element_type=jnp.float32) + b2_ref[...]
        a_ref[...] = (jnp.dot(h2, wa_ref[...],
                              preferred_element_type=jnp.float32)
                      + jnp.dot(s_ref[...], wsa_ref[...],
                                preferred_element_type=jnp.float32)
                      + bl1_ref[...])
        b_ref[...] = (jnp.dot(h2, wb_ref[...],
                              preferred_element_type=jnp.float32)
                      + jnp.dot(s_ref[...], wsb_ref[...],
                                preferred_element_type=jnp.float32))


def _dense_call(adj, x, s, W1, b1r, W2, b2r, wa, wsa, wb, wsb, bl1r):
    const = pl.BlockSpec((NHID, NHID), lambda l, i: (0, 0))
    consts16 = pl.BlockSpec((NS, NHID), lambda l, i: (0, 0))
    row = pl.BlockSpec((1, NHID), lambda l, i: (0, 0))
    return pl.pallas_call(
        _dense_body,
        grid=(2, N // BM),
        in_specs=[pl.BlockSpec((BM, N), lambda l, i: (i, 0)),
                  pl.BlockSpec((N, NFEAT), lambda l, i: (0, 0)),
                  pl.BlockSpec((BM, NS), lambda l, i: (i, 0)),
                  pl.BlockSpec((NFEAT, NHID), lambda l, i: (0, 0)),
                  row, const, row, const, consts16, const, consts16, row],
        out_specs=[pl.BlockSpec((BM, NHID), lambda l, i: (l * i, 0)),
                   pl.BlockSpec((BM, NHID), lambda l, i: (l * i, 0))],
        out_shape=[jax.ShapeDtypeStruct((N, NHID), jnp.float32),
                   jax.ShapeDtypeStruct((N, NHID), jnp.float32)],
        scratch_shapes=[pltpu.VMEM((N, NHID), jnp.float32)],
        compiler_params=pltpu.CompilerParams(
            dimension_semantics=("arbitrary", "arbitrary")),
    )(adj, x, s, W1, b1r, W2, b2r, wa, wsa, wb, wsb, bl1r)


_NC = 2
_NSUB = 16
_NW = _NC * _NSUB
_C = 128
_PERW = E // _NW
_NCT = -(-_PERW // _C)


def _take(v, idx):
    return jnp.take_along_axis(v, idx, axis=0)


def _edge_body(a_hbm, b_hbm, src_hbm, dst_hbm, wl2_hbm, out_hbm,
               si_v, di_v, ar_v, br_v, o_v, w_v,
               sa0, sa1, sb0, sb1, so0, so1):
    wid = lax.axis_index("s") * _NC + lax.axis_index("c")
    pltpu.sync_copy(wl2_hbm, w_v)
    ebase = wid * _PERW
    pltpu.sync_copy(src_hbm.at[pl.ds(ebase, _PERW)], si_v)
    pltpu.sync_copy(dst_hbm.at[pl.ds(ebase, _PERW)], di_v)
    nct = _NCT
    sem_a = (sa0, sa1)
    sem_b = (sb0, sb1)
    sem_o = (so0, so1)

    nfc = NHID // 16
    wv = [w_v[pl.ds(fc * 16, 16)] for fc in range(nfc)]
    lane = lax.iota(jnp.int32, 16)
    perm = {s: lane ^ s for s in (8, 4, 2, 1)}
    mask = {s: (lane & s) == 0 for s in (8, 4, 2, 1)}

    def _merge(a, b, st):
        a2 = a + _take(a, perm[st])
        b2 = b + _take(b, perm[st])
        return jnp.where(mask[st], a2, b2)

    def _hsum16(vs):
        for st in (8, 4, 2, 1):
            half = len(vs) // 2
            vs = [_merge(vs[i], vs[i + half], st) for i in range(half)]
        return vs[0]

    def _chunk_off(t):
        return jnp.minimum(t * _C, _PERW - _C)

    _H = _C // 2

    def _fetch(t, slot):
        off = _chunk_off(t)
        pltpu.async_copy(a_hbm.at[si_v.at[pl.ds(off, _H)]],
                         ar_v.at[slot, pl.ds(0, _H)], sem_a[slot])
        pltpu.async_copy(a_hbm.at[si_v.at[pl.ds(off + _H, _H)]],
                         ar_v.at[slot, pl.ds(_H, _H)], sem_a[slot])
        pltpu.async_copy(b_hbm.at[di_v.at[pl.ds(off, _H)]],
                         br_v.at[slot, pl.ds(0, _H)], sem_b[slot])
        pltpu.async_copy(b_hbm.at[di_v.at[pl.ds(off + _H, _H)]],
                         br_v.at[slot, pl.ds(_H, _H)], sem_b[slot])

    def _wait_rows(slot):
        for h in (0, _H):
            pltpu.make_async_copy(a_hbm.at[si_v.at[pl.ds(h, _H)]],
                                  ar_v.at[slot, pl.ds(h, _H)],
                                  sem_a[slot]).wait()
            pltpu.make_async_copy(b_hbm.at[di_v.at[pl.ds(h, _H)]],
                                  br_v.at[slot, pl.ds(h, _H)],
                                  sem_b[slot]).wait()

    def _drain_out(slot, t):
        pltpu.make_async_copy(o_v.at[slot],
                              out_hbm.at[pl.ds(ebase + _chunk_off(t), _C)],
                              sem_o[slot]).wait()

    def _compute(t, slot):
        _wait_rows(slot)

        @pl.when(t >= 2)
        def _():
            _drain_out(slot, t - 2)

        def group_body(g, c2):
            vs = []
            for l in range(16):
                e = g * 16 + l
                acc = jnp.zeros((16,), jnp.float32)
                for fc in range(nfc):
                    va = ar_v[slot, e, pl.ds(fc * 16, 16)]
                    vb = br_v[slot, e, pl.ds(fc * 16, 16)]
                    acc = acc + jnp.maximum(va + vb, 0.0) * wv[fc]
                vs.append(acc)
            o_v[slot, pl.ds(g * 16, 16)] = _hsum16(vs)
            return c2

        lax.fori_loop(0, _C // 16, group_body, 0)

        @pl.when(t + 2 < nct)
        def _():
            _fetch(t + 2, slot)

        pltpu.async_copy(o_v.at[slot],
                         out_hbm.at[pl.ds(ebase + _chunk_off(t), _C)],
                         sem_o[slot])

    _fetch(0, 0)
    _fetch(1, 1)

    def pair_body(p, carry):
        t0 = 2 * p
        _compute(t0, 0)
        _compute(t0 + 1, 1)
        return carry

    lax.fori_loop(0, nct // 2, pair_body, 0)
    _drain_out(0, 0)
    _drain_out(1, 0)


def _edge_call(a, b, src, dst, wl2):
    mesh = plsc.VectorSubcoreMesh(core_axis_name="c", subcore_axis_name="s")
    f = functools.partial(
        pl.kernel,
        out_type=jax.ShapeDtypeStruct((E,), jnp.float32),
        mesh=mesh,
        scratch_types=[
            pltpu.VMEM((_PERW,), jnp.int32),
            pltpu.VMEM((_PERW,), jnp.int32),
            pltpu.VMEM((2, _C, NHID), jnp.float32),
            pltpu.VMEM((2, _C, NHID), jnp.float32),
            pltpu.VMEM((2, _C), jnp.float32),
            pltpu.VMEM((NHID,), jnp.float32),
            pltpu.SemaphoreType.DMA,
            pltpu.SemaphoreType.DMA,
            pltpu.SemaphoreType.DMA,
            pltpu.SemaphoreType.DMA,
            pltpu.SemaphoreType.DMA,
            pltpu.SemaphoreType.DMA,
        ],
    )(_edge_body)
    return f(a, b, src, dst, wl2)


def kernel(s, x, adj, train_ids, W1, b1, W2, b2, Wl1, bl1, Wl2, bl2):
    b1r = b1.reshape(1, NHID)
    b2r = b2.reshape(1, NHID)
    bl1r = bl1.reshape(1, NHID)
    wa = Wl1[0:NHID]
    wsa = Wl1[NHID:NHID + NS]
    wb = Wl1[NHID + NS:2 * NHID + NS]
    wsb = Wl1[2 * NHID + NS:]
    src = train_ids[:, 0].astype(jnp.int32)
    dst = train_ids[:, 1].astype(jnp.int32)

    a, b = _dense_call(adj, x, s, W1, b1r, W2, b2r, wa, wsa, wb, wsb, bl1r)
    out = _edge_call(a, b, src, dst, Wl2.reshape(NHID))
    return out + bl2[0]
